# baseline (speedup 1.0000x reference)
import jax
import jax.numpy as jnp
from jax.experimental import pallas as pl

_NF = 26


def kernel(x, tables, W1, b1, gamma1, beta1, W2, b2, gamma2, beta2, W3, b3):
    emb = jnp.concatenate(
        [jnp.take(tables[i], x[:, i], axis=0) for i in range(_NF)], axis=1)
    h = jax.nn.relu(emb @ W1 + b1)
    mean = jnp.mean(h, axis=0, keepdims=True)
    var = jnp.var(h, axis=0, keepdims=True)
    h = (h - mean) / jnp.sqrt(var + 1e-5) * gamma1 + beta1
    h = jax.nn.relu(h @ W2 + b2)
    mean = jnp.mean(h, axis=0, keepdims=True)
    var = jnp.var(h, axis=0, keepdims=True)
    h = (h - mean) / jnp.sqrt(var + 1e-5) * gamma2 + beta2
    out = h @ W3 + b3
    return jnp.squeeze(out, axis=-1)


# speedup vs baseline: 2.0104x; 2.0104x over previous
import functools

import jax
import jax.numpy as jnp
from jax import lax
from jax.experimental import pallas as pl
from jax.experimental.pallas import tpu as pltpu
from jax.experimental.pallas import tpu_sc as plsc

_V = 100000
_NF = 26
_D = 32
_B = 16384
_EM = _NF * _D
_H1 = 128
_H2 = 64

_NC = 2
_NS = 16
_NW = _NC * _NS
_ROWS = _B * _NF
_BPW = _ROWS // _NW
_CH = 832
_NT = _BPW // _CH


def _make_gather():
    mesh = plsc.VectorSubcoreMesh(core_axis_name="c", subcore_axis_name="s")

    @functools.partial(
        pl.kernel,
        mesh=mesh,
        out_type=jax.ShapeDtypeStruct((_ROWS, _D), jnp.float32),
        scratch_types=[
            pltpu.VMEM((_BPW,), jnp.int32),
            pltpu.VMEM((_CH, _D), jnp.float32),
            pltpu.VMEM((_CH, _D), jnp.float32),
            pltpu.SemaphoreType.DMA,
            pltpu.SemaphoreType.DMA,
        ],
        compiler_params=pltpu.CompilerParams(use_tc_tiling_on_sc=False),
    )
    def gather_k(table_hbm, idx_hbm, out_hbm, idx_v, buf0, buf1, sem0, sem1):
        wid = lax.axis_index("s") * _NC + lax.axis_index("c")
        base = wid * _BPW
        pltpu.sync_copy(idx_hbm.at[pl.ds(base, _BPW)], idx_v)
        bufs = (buf0, buf1)
        sems = (sem0, sem1)
        prev = pltpu.async_copy(table_hbm.at[idx_v.at[pl.ds(0, _CH)]], buf0, sem0)
        for t in range(1, _NT):
            cur = pltpu.async_copy(
                table_hbm.at[idx_v.at[pl.ds(t * _CH, _CH)]], bufs[t % 2], sems[t % 2])
            prev.wait()
            pltpu.sync_copy(bufs[(t - 1) % 2],
                            out_hbm.at[pl.ds(base + (t - 1) * _CH, _CH)])
            prev = cur
        prev.wait()
        pltpu.sync_copy(bufs[(_NT - 1) % 2],
                        out_hbm.at[pl.ds(base + (_NT - 1) * _CH, _CH)])

    return gather_k


_gather_cache = []


def _gather(table_flat, idx):
    if not _gather_cache:
        _gather_cache.append(_make_gather())
    return _gather_cache[0](table_flat, idx)

_BB = 2048
_NB = _B // _BB


def _mlp_body(emb_ref, w1_ref, b1_ref, g1_ref, be1_ref, w2_ref, b2_ref,
              g2_ref, be2_ref, w3_ref, b3_ref, out_ref, h1_scr, s1_scr, ss1_scr):
    j = pl.program_id(0)
    h1 = jnp.dot(emb_ref[...], w1_ref[...], preferred_element_type=jnp.float32)
    h1 = jnp.maximum(h1 + b1_ref[...], 0.0)
    h1_scr[pl.ds(j * _BB, _BB), :] = h1

    @pl.when(j == 0)
    def _init():
        s1_scr[...] = jnp.zeros_like(s1_scr)
        ss1_scr[...] = jnp.zeros_like(ss1_scr)

    s1_scr[...] += jnp.sum(h1, axis=0, keepdims=True)
    ss1_scr[...] += jnp.sum(h1 * h1, axis=0, keepdims=True)

    @pl.when(j == _NB - 1)
    def _tail():
        inv_b = 1.0 / _B
        mean1 = s1_scr[...] * inv_b
        var1 = ss1_scr[...] * inv_b - mean1 * mean1
        h1n = ((h1_scr[...] - mean1) * lax.rsqrt(var1 + 1e-5)
               * g1_ref[...] + be1_ref[...])
        h2 = jnp.dot(h1n, w2_ref[...], preferred_element_type=jnp.float32)
        h2 = jnp.maximum(h2 + b2_ref[...], 0.0)
        mean2 = jnp.mean(h2, axis=0, keepdims=True)
        var2 = jnp.mean(h2 * h2, axis=0, keepdims=True) - mean2 * mean2
        h2n = (h2 - mean2) * lax.rsqrt(var2 + 1e-5) * g2_ref[...] + be2_ref[...]
        out_ref[...] = (jnp.dot(h2n, w3_ref[...],
                                preferred_element_type=jnp.float32) + b3_ref[...])


def _make_mlp():
    const = lambda j: (0, 0)
    return pl.pallas_call(
        _mlp_body,
        grid=(_NB,),
        in_specs=[
            pl.BlockSpec((_BB, _EM), lambda j: (j, 0)),
            pl.BlockSpec((_EM, _H1), const),
            pl.BlockSpec((1, _H1), const),
            pl.BlockSpec((1, _H1), const),
            pl.BlockSpec((1, _H1), const),
            pl.BlockSpec((_H1, _H2), const),
            pl.BlockSpec((1, _H2), const),
            pl.BlockSpec((1, _H2), const),
            pl.BlockSpec((1, _H2), const),
            pl.BlockSpec((_H2, 1), const),
            pl.BlockSpec((1, 1), const),
        ],
        out_specs=pl.BlockSpec((_B, 1), const),
        out_shape=jax.ShapeDtypeStruct((_B, 1), jnp.float32),
        scratch_shapes=[
            pltpu.VMEM((_B, _H1), jnp.float32),
            pltpu.VMEM((1, _H1), jnp.float32),
            pltpu.VMEM((1, _H1), jnp.float32),
        ],
    )


_mlp = _make_mlp()


def kernel(x, tables, W1, b1, gamma1, beta1, W2, b2, gamma2, beta2, W3, b3):
    table_flat = tables.reshape(_NF * _V, _D)
    idx = (x.astype(jnp.int32)
           + jnp.arange(_NF, dtype=jnp.int32)[None, :] * _V).reshape(-1)
    emb = _gather(table_flat, idx).reshape(_B, _EM)
    out = _mlp(emb, W1,
               b1.reshape(1, _H1), gamma1.reshape(1, _H1), beta1.reshape(1, _H1),
               W2, b2.reshape(1, _H2), gamma2.reshape(1, _H2), beta2.reshape(1, _H2),
               W3, b3.reshape(1, 1))
    return out.reshape(_B)


# speedup vs baseline: 2.2113x; 1.1000x over previous
import functools

import jax
import jax.numpy as jnp
from jax import lax
from jax.experimental import pallas as pl
from jax.experimental.pallas import tpu as pltpu
from jax.experimental.pallas import tpu_sc as plsc

_V = 100000
_NF = 26
_D = 32
_B = 16384
_EM = _NF * _D
_H1 = 128
_H2 = 64
_Q = _V // 4

_GROUPS = (7, 7, 6, 6)


def _repack_body(in_ref, eye_ref, out_ref):
    a = in_ref[0]
    eye = eye_ref[...]
    vc = 12500
    for t in range(_V // vc):
        c, rb = t // (_Q // vc), (t % (_Q // vc)) * vc
        bt = lax.dot_general(a[:, t * vc:(t + 1) * vc], eye,
                             (((0,), (0,)), ((), ())),
                             preferred_element_type=jnp.float32)
        out_ref[rb:rb + vc, c * _D:(c + 1) * _D] = bt


@functools.cache
def _make_repack(nf):
    return pl.pallas_call(
        _repack_body,
        grid=(nf,),
        in_specs=[pl.BlockSpec((1, _D, _V), lambda f: (f, 0, 0)),
                  pl.BlockSpec((_D, _D), lambda f: (0, 0))],
        out_specs=pl.BlockSpec((_Q, 128), lambda f: (f, 0)),
        out_shape=jax.ShapeDtypeStruct((nf * _Q, 128), jnp.float32),
        compiler_params=pltpu.CompilerParams(
            vmem_limit_bytes=128 * 1024 * 1024),
    )


_NC = 2
_NS = 16
_NW = _NC * _NS


@functools.cache
def _make_gather(nf):
    rows = _B * nf
    bpw = rows // _NW
    ch = 512
    nt = bpw // ch
    assert bpw % ch == 0 and rows % _NW == 0
    mesh = plsc.VectorSubcoreMesh(core_axis_name="c", subcore_axis_name="s")

    @functools.partial(
        pl.kernel,
        mesh=mesh,
        out_type=jax.ShapeDtypeStruct((rows, _D), jnp.float32),
        scratch_types=[
            pltpu.VMEM((bpw,), jnp.int32),
            pltpu.VMEM((ch, _D), jnp.float32),
            pltpu.VMEM((ch, _D), jnp.float32),
            pltpu.SemaphoreType.DMA,
            pltpu.SemaphoreType.DMA,
        ],
        compiler_params=pltpu.CompilerParams(use_tc_tiling_on_sc=False),
    )
    def gather_k(table_hbm, idx_hbm, out_hbm, idx_v, buf0, buf1, sem0, sem1):
        wid = lax.axis_index("s") * _NC + lax.axis_index("c")
        base = wid * bpw
        pltpu.sync_copy(idx_hbm.at[pl.ds(base, bpw)], idx_v)
        bufs = (buf0, buf1)
        sems = (sem0, sem1)
        prev = pltpu.async_copy(table_hbm.at[idx_v.at[pl.ds(0, ch)]], buf0, sem0)
        for t in range(1, nt):
            cur = pltpu.async_copy(
                table_hbm.at[idx_v.at[pl.ds(t * ch, ch)]], bufs[t % 2], sems[t % 2])
            prev.wait()
            pltpu.sync_copy(bufs[(t - 1) % 2],
                            out_hbm.at[pl.ds(base + (t - 1) * ch, ch)])
            prev = cur
        prev.wait()
        pltpu.sync_copy(bufs[(nt - 1) % 2],
                        out_hbm.at[pl.ds(base + (nt - 1) * ch, ch)])

    return gather_k


_BB = 2048
_NB = _B // _BB


def _mlp_body(e0_ref, e1_ref, e2_ref, e3_ref, w10_ref, w11_ref, w12_ref,
              w13_ref, b1_ref, g1_ref, be1_ref, w2_ref, b2_ref,
              g2_ref, be2_ref, w3_ref, b3_ref, out_ref, h1_scr, s1_scr, ss1_scr):
    j = pl.program_id(0)
    h1 = lax.dot_general(e0_ref[...], w10_ref[...], (((1,), (0,)), ((), ())),
                         preferred_element_type=jnp.float32)
    for e_ref, w_ref in ((e1_ref, w11_ref), (e2_ref, w12_ref), (e3_ref, w13_ref)):
        h1 = h1 + lax.dot_general(e_ref[...], w_ref[...],
                                  (((1,), (0,)), ((), ())),
                                  preferred_element_type=jnp.float32)
    h1 = jnp.maximum(h1 + b1_ref[...], 0.0)
    h1_scr[pl.ds(j * _BB, _BB), :] = h1

    @pl.when(j == 0)
    def _init():
        s1_scr[...] = jnp.zeros_like(s1_scr)
        ss1_scr[...] = jnp.zeros_like(ss1_scr)

    s1_scr[...] += jnp.sum(h1, axis=0, keepdims=True)
    ss1_scr[...] += jnp.sum(h1 * h1, axis=0, keepdims=True)

    @pl.when(j == _NB - 1)
    def _tail():
        inv_b = 1.0 / _B
        mean1 = s1_scr[...] * inv_b
        var1 = ss1_scr[...] * inv_b - mean1 * mean1
        h1n = ((h1_scr[...] - mean1) * lax.rsqrt(var1 + 1e-5)
               * g1_ref[...] + be1_ref[...])
        h2 = jnp.dot(h1n, w2_ref[...], preferred_element_type=jnp.float32)
        h2 = jnp.maximum(h2 + b2_ref[...], 0.0)
        mean2 = jnp.mean(h2, axis=0, keepdims=True)
        var2 = jnp.mean(h2 * h2, axis=0, keepdims=True) - mean2 * mean2
        h2n = (h2 - mean2) * lax.rsqrt(var2 + 1e-5) * g2_ref[...] + be2_ref[...]
        out_ref[...] = (jnp.dot(h2n, w3_ref[...],
                                preferred_element_type=jnp.float32) + b3_ref[...])


def _make_mlp():
    const = lambda j: (0, 0)
    in_specs = []
    for nf in _GROUPS:
        in_specs.append(pl.BlockSpec((_BB, nf * _D), lambda j: (j, 0)))
    for nf in _GROUPS:
        in_specs.append(pl.BlockSpec((nf * _D, _H1), const))
    in_specs += [
        pl.BlockSpec((1, _H1), const),
        pl.BlockSpec((1, _H1), const),
        pl.BlockSpec((1, _H1), const),
        pl.BlockSpec((_H1, _H2), const),
        pl.BlockSpec((1, _H2), const),
        pl.BlockSpec((1, _H2), const),
        pl.BlockSpec((1, _H2), const),
        pl.BlockSpec((_H2, 1), const),
        pl.BlockSpec((1, 1), const),
    ]
    return pl.pallas_call(
        _mlp_body,
        grid=(_NB,),
        in_specs=in_specs,
        out_specs=pl.BlockSpec((_B, 1), const),
        out_shape=jax.ShapeDtypeStruct((_B, 1), jnp.float32),
        scratch_shapes=[
            pltpu.VMEM((_B, _H1), jnp.float32),
            pltpu.VMEM((1, _H1), jnp.float32),
            pltpu.VMEM((1, _H1), jnp.float32),
        ],
    )


_mlp_cache = []


def kernel(x, tables, W1, b1, gamma1, beta1, W2, b2, gamma2, beta2, W3, b3):
    tt = jnp.transpose(tables, (0, 2, 1))
    eye = jnp.eye(_D, dtype=jnp.float32)
    xi = x.astype(jnp.int32)

    embs, w1s = [], []
    f0 = 0
    for nf in _GROUPS:
        packed = _make_repack(nf)(tt[f0:f0 + nf], eye)
        table_flat = packed.reshape(nf * _V, _D)
        xg = xi[:, f0:f0 + nf]
        idx = (jnp.arange(nf, dtype=jnp.int32)[None, :] * _V
               + (xg % _Q) * 4 + xg // _Q).reshape(-1)
        embs.append(_make_gather(nf)(table_flat, idx).reshape(_B, nf * _D))
        w1s.append(W1[f0 * _D:(f0 + nf) * _D])
        f0 += nf

    if not _mlp_cache:
        _mlp_cache.append(_make_mlp())
    out = _mlp_cache[0](
        *embs, *w1s,
        b1.reshape(1, _H1), gamma1.reshape(1, _H1), beta1.reshape(1, _H1),
        W2, b2.reshape(1, _H2), gamma2.reshape(1, _H2), beta2.reshape(1, _H2),
        W3, b3.reshape(1, 1))
    return out.reshape(_B)


# speedup vs baseline: 6.5864x; 2.9785x over previous
import functools

import jax
import jax.numpy as jnp
from jax import lax
from jax.experimental import pallas as pl
from jax.experimental.pallas import tpu as pltpu
from jax.experimental.pallas import tpu_sc as plsc

_V = 100000
_NF = 26
_D = 32
_B = 16384
_EM = _NF * _D
_H1 = 128
_H2 = 64

_NC = 2
_NS = 16
_NW = _NC * _NS
_ROWS = _B * _NF
_BPW = _ROWS // _NW
_CH = 832
_NT = _BPW // _CH


def _make_gather():
    mesh = plsc.VectorSubcoreMesh(core_axis_name="c", subcore_axis_name="s")

    @functools.partial(
        pl.kernel,
        mesh=mesh,
        out_type=jax.ShapeDtypeStruct((_ROWS, _D), jnp.float32),
        scratch_types=[
            pltpu.VMEM((_BPW,), jnp.int32),
            pltpu.VMEM((_CH, _D), jnp.float32),
            pltpu.VMEM((_CH, _D), jnp.float32),
            pltpu.SemaphoreType.DMA,
            pltpu.SemaphoreType.DMA,
        ],
        compiler_params=pltpu.CompilerParams(use_tc_tiling_on_sc=False),
    )
    def gather_k(table_hbm, idx_hbm, out_hbm, idx_v, buf0, buf1, sem0, sem1):
        wid = lax.axis_index("s") * _NC + lax.axis_index("c")
        base = wid * _BPW
        pltpu.sync_copy(idx_hbm.at[pl.ds(base, _BPW)], idx_v)
        bufs = (buf0, buf1)
        sems = (sem0, sem1)
        prev = pltpu.async_copy(table_hbm.at[idx_v.at[pl.ds(0, _CH)]], buf0, sem0)
        for t in range(1, _NT):
            cur = pltpu.async_copy(
                table_hbm.at[idx_v.at[pl.ds(t * _CH, _CH)]], bufs[t % 2], sems[t % 2])
            prev.wait()
            pltpu.sync_copy(bufs[(t - 1) % 2],
                            out_hbm.at[pl.ds(base + (t - 1) * _CH, _CH)])
            prev = cur
        prev.wait()
        pltpu.sync_copy(bufs[(_NT - 1) % 2],
                        out_hbm.at[pl.ds(base + (_NT - 1) * _CH, _CH)])

    return gather_k


_gather_cache = []


def _gather(table_flat, idx):
    if not _gather_cache:
        _gather_cache.append(_make_gather())
    return _gather_cache[0](table_flat, idx)


_Q = _V // 4


def _repack_body(in_ref, eye_ref, out_ref):
    a = in_ref[0]
    eye = eye_ref[...]
    hb = 1000
    for h in range(_Q // hb):
        a4 = jnp.concatenate(
            [a[:, c * _Q + h * hb:c * _Q + (h + 1) * hb] for c in range(4)],
            axis=0)
        bt = lax.dot_general(a4, eye, (((0,), (0,)), ((), ())),
                             preferred_element_type=jnp.float32)
        out_ref[h * hb:(h + 1) * hb, :] = bt


def _make_repack():
    return pl.pallas_call(
        _repack_body,
        grid=(_NF,),
        in_specs=[pl.BlockSpec((1, _D, _V), lambda f: (f, 0, 0)),
                  pl.BlockSpec((128, 128), lambda f: (0, 0))],
        out_specs=pl.BlockSpec((_Q, 128), lambda f: (f, 0)),
        out_shape=jax.ShapeDtypeStruct((_NF * _Q, 128), jnp.float32),
        compiler_params=pltpu.CompilerParams(
            vmem_limit_bytes=128 * 1024 * 1024),
    )


_repack = _make_repack()

_BB = 2048
_NB = _B // _BB


def _mlp_body(emb_ref, w1_ref, b1_ref, g1_ref, be1_ref, w2_ref, b2_ref,
              g2_ref, be2_ref, w3_ref, b3_ref, out_ref, h1_scr, s1_scr, ss1_scr):
    j = pl.program_id(0)
    h1 = jnp.dot(emb_ref[...], w1_ref[...], preferred_element_type=jnp.float32)
    h1 = jnp.maximum(h1 + b1_ref[...], 0.0)
    h1_scr[pl.ds(j * _BB, _BB), :] = h1

    @pl.when(j == 0)
    def _init():
        s1_scr[...] = jnp.zeros_like(s1_scr)
        ss1_scr[...] = jnp.zeros_like(ss1_scr)

    s1_scr[...] += jnp.sum(h1, axis=0, keepdims=True)
    ss1_scr[...] += jnp.sum(h1 * h1, axis=0, keepdims=True)

    @pl.when(j == _NB - 1)
    def _tail():
        inv_b = 1.0 / _B
        mean1 = s1_scr[...] * inv_b
        var1 = ss1_scr[...] * inv_b - mean1 * mean1
        h1n = ((h1_scr[...] - mean1) * lax.rsqrt(var1 + 1e-5)
               * g1_ref[...] + be1_ref[...])
        h2 = jnp.dot(h1n, w2_ref[...], preferred_element_type=jnp.float32)
        h2 = jnp.maximum(h2 + b2_ref[...], 0.0)
        mean2 = jnp.mean(h2, axis=0, keepdims=True)
        var2 = jnp.mean(h2 * h2, axis=0, keepdims=True) - mean2 * mean2
        h2n = (h2 - mean2) * lax.rsqrt(var2 + 1e-5) * g2_ref[...] + be2_ref[...]
        out_ref[...] = (jnp.dot(h2n, w3_ref[...],
                                preferred_element_type=jnp.float32) + b3_ref[...])


def _make_mlp():
    const = lambda j: (0, 0)
    return pl.pallas_call(
        _mlp_body,
        grid=(_NB,),
        in_specs=[
            pl.BlockSpec((_BB, _EM), lambda j: (j, 0)),
            pl.BlockSpec((_EM, _H1), const),
            pl.BlockSpec((1, _H1), const),
            pl.BlockSpec((1, _H1), const),
            pl.BlockSpec((1, _H1), const),
            pl.BlockSpec((_H1, _H2), const),
            pl.BlockSpec((1, _H2), const),
            pl.BlockSpec((1, _H2), const),
            pl.BlockSpec((1, _H2), const),
            pl.BlockSpec((_H2, 1), const),
            pl.BlockSpec((1, 1), const),
        ],
        out_specs=pl.BlockSpec((_B, 1), const),
        out_shape=jax.ShapeDtypeStruct((_B, 1), jnp.float32),
        scratch_shapes=[
            pltpu.VMEM((_B, _H1), jnp.float32),
            pltpu.VMEM((1, _H1), jnp.float32),
            pltpu.VMEM((1, _H1), jnp.float32),
        ],
    )


_mlp = _make_mlp()


def kernel(x, tables, W1, b1, gamma1, beta1, W2, b2, gamma2, beta2, W3, b3):
    tt = jnp.transpose(tables, (0, 2, 1))
    eye = jnp.eye(128, dtype=jnp.float32)
    table_flat = _repack(tt, eye).reshape(_NF * _V, _D)
    xi = x.astype(jnp.int32)
    idx = (jnp.arange(_NF, dtype=jnp.int32)[None, :] * _V
           + (xi % _Q) * 4 + xi // _Q).reshape(-1)
    emb = _gather(table_flat, idx).reshape(_B, _EM)
    out = _mlp(emb, W1,
               b1.reshape(1, _H1), gamma1.reshape(1, _H1), beta1.reshape(1, _H1),
               W2, b2.reshape(1, _H2), gamma2.reshape(1, _H2), beta2.reshape(1, _H2),
               W3, b3.reshape(1, 1))
    return out.reshape(_B)
